# baseline (speedup 1.0000x reference)
import functools

import jax
import jax.numpy as jnp
from jax import lax
from jax.experimental import pallas as pl
from jax.experimental.pallas import tpu as pltpu
from jax.experimental.pallas import tpu_sc as plsc

_SEQ = 4096
_TAB = 2048
_W = 8192
_NEG = -1e9


def _rpe_body(neg_hbm, tab_hbm, out_hbm, tab_v, w2_v, w2_sh, sem, fill_sem):
    c = lax.axis_index("c")
    s = lax.axis_index("s")
    wid = c * 16 + s

    pltpu.async_copy(neg_hbm.at[:, pl.ds(0, 2032)], w2_v.at[:, pl.ds(0, 2032)],
                     fill_sem)
    pltpu.async_copy(neg_hbm.at[:, pl.ds(0, 4064)],
                     w2_v.at[:, pl.ds(4128, 4064)], fill_sem)
    pltpu.sync_copy(tab_hbm, tab_v)

    def place(jc, carry):
        v = tab_v[pl.ds(jc * 16, 16)]
        rv = lax.rev(v, (0,))
        base = 4080 - jc * 16
        for r in range(8):
            w2_v[r, pl.ds(base + r, 16)] = rv
        return carry

    lax.fori_loop(0, 128, place, 0)

    neg16 = jnp.full((16,), _NEG, dtype=jnp.float32)
    for r in range(8):
        w2_v[r, pl.ds(2048 + r - 15, 16)] = neg16
        w2_v[r, pl.ds(2032, 16)] = neg16

    pltpu.make_async_copy(neg_hbm.at[:, pl.ds(0, 2032)],
                          w2_v.at[:, pl.ds(0, 2032)], fill_sem).wait()
    pltpu.make_async_copy(neg_hbm.at[:, pl.ds(0, 4064)],
                          w2_v.at[:, pl.ds(4128, 4064)], fill_sem).wait()
    for r in range(8):
        w2_v[r, pl.ds(4096 + r, 16)] = neg16
        w2_v[r, pl.ds(4112 + r, 16)] = neg16

    @pl.when(s < 8)
    def _publish():
        pltpu.sync_copy(w2_v.at[pl.ds(s, 1), :], w2_sh.at[pl.ds(s, 1), :])

    plsc.subcore_barrier()

    base_row = wid * 128

    @pl.when(s % 2 == 0)
    def _from_local():
        def fire(g, carry):
            i0 = base_row + g * 8
            pltpu.async_copy(
                w2_v.at[:, pl.ds(4096 - i0, 4096)],
                out_hbm.at[pl.ds(i0, 8), :],
                sem,
            )
            return carry

        lax.fori_loop(0, 16, fire, 0)

        def drain(g, carry):
            i0 = base_row + g * 8
            pltpu.make_async_copy(
                w2_v.at[:, pl.ds(4096 - i0, 4096)],
                out_hbm.at[pl.ds(i0, 8), :],
                sem,
            ).wait()
            return carry

        lax.fori_loop(0, 16, drain, 0)

    @pl.when(s % 2 == 1)
    def _from_shared():
        def fire(g, carry):
            i0 = base_row + g * 8
            pltpu.async_copy(
                w2_sh.at[:, pl.ds(4096 - i0, 4096)],
                out_hbm.at[pl.ds(i0, 8), :],
                sem,
            )
            return carry

        lax.fori_loop(0, 16, fire, 0)

        def drain(g, carry):
            i0 = base_row + g * 8
            pltpu.make_async_copy(
                w2_sh.at[:, pl.ds(4096 - i0, 4096)],
                out_hbm.at[pl.ds(i0, 8), :],
                sem,
            ).wait()
            return carry

        lax.fori_loop(0, 16, drain, 0)


_rpe = functools.partial(
    pl.kernel,
    out_type=jax.ShapeDtypeStruct((_SEQ, _SEQ), jnp.float32),
    mesh=plsc.VectorSubcoreMesh(core_axis_name="c", subcore_axis_name="s"),
    scratch_types=[
        pltpu.VMEM((_TAB,), jnp.float32),
        pltpu.VMEM((8, _W), jnp.float32),
        pltpu.VMEM_SHARED((8, _W), jnp.float32),
        pltpu.SemaphoreType.DMA,
        pltpu.SemaphoreType.DMA,
    ],
    compiler_params=pltpu.CompilerParams(use_tc_tiling_on_sc=False),
)(_rpe_body)


def kernel(seq, table):
    del seq
    tab = table.reshape(_TAB)
    neg = jnp.full((8, 4064), _NEG, dtype=jnp.float32)
    out = _rpe(neg, tab)
    return out[:, :, None]


# speedup vs baseline: 1.1647x; 1.1647x over previous
import functools

import jax
import jax.numpy as jnp
from jax import lax
from jax.experimental import pallas as pl
from jax.experimental.pallas import tpu as pltpu
from jax.experimental.pallas import tpu_sc as plsc

_SEQ = 4096
_TAB = 2048
_W = 8192
_NEG = -1e9


def _rpe_body(neg_hbm, tab_hbm, out_hbm, tab_v, w2_v, sem, fill_sem):
    c = lax.axis_index("c")
    s = lax.axis_index("s")
    wid = c * 16 + s

    pltpu.async_copy(neg_hbm.at[:, pl.ds(0, 2032)], w2_v.at[:, pl.ds(0, 2032)],
                     fill_sem)
    pltpu.async_copy(neg_hbm.at[:, pl.ds(0, 4064)],
                     w2_v.at[:, pl.ds(4128, 4064)], fill_sem)
    pltpu.sync_copy(tab_hbm, tab_v)

    def place(jc, carry):
        v = tab_v[pl.ds(jc * 16, 16)]
        rv = lax.rev(v, (0,))
        base = 4080 - jc * 16
        for r in range(8):
            w2_v[r, pl.ds(base + r, 16)] = rv
        return carry

    lax.fori_loop(0, 128, place, 0)

    neg16 = jnp.full((16,), _NEG, dtype=jnp.float32)
    for r in range(8):
        w2_v[r, pl.ds(2048 + r - 15, 16)] = neg16
        w2_v[r, pl.ds(2032, 16)] = neg16

    pltpu.make_async_copy(neg_hbm.at[:, pl.ds(0, 2032)],
                          w2_v.at[:, pl.ds(0, 2032)], fill_sem).wait()
    pltpu.make_async_copy(neg_hbm.at[:, pl.ds(0, 4064)],
                          w2_v.at[:, pl.ds(4128, 4064)], fill_sem).wait()
    for r in range(8):
        w2_v[r, pl.ds(4096 + r, 16)] = neg16
        w2_v[r, pl.ds(4112 + r, 16)] = neg16

    base_row = wid * 128

    def fire(g, carry):
        i0 = base_row + g * 8
        pltpu.async_copy(
            w2_v.at[:, pl.ds(4096 - i0, 4096)],
            out_hbm.at[pl.ds(i0, 8), :],
            sem,
        )
        return carry

    lax.fori_loop(0, 16, fire, 0)

    def drain(g, carry):
        i0 = base_row + g * 8
        pltpu.make_async_copy(
            w2_v.at[:, pl.ds(4096 - i0, 4096)],
            out_hbm.at[pl.ds(i0, 8), :],
            sem,
        ).wait()
        return carry

    lax.fori_loop(0, 16, drain, 0)


_rpe = functools.partial(
    pl.kernel,
    out_type=jax.ShapeDtypeStruct((_SEQ, _SEQ), jnp.float32),
    mesh=plsc.VectorSubcoreMesh(core_axis_name="c", subcore_axis_name="s"),
    scratch_types=[
        pltpu.VMEM((_TAB,), jnp.float32),
        pltpu.VMEM((8, _W), jnp.float32),
        pltpu.SemaphoreType.DMA,
        pltpu.SemaphoreType.DMA,
    ],
    compiler_params=pltpu.CompilerParams(use_tc_tiling_on_sc=False),
)(_rpe_body)


def kernel(seq, table):
    del seq
    tab = table.reshape(_TAB)
    neg = jnp.full((8, 4064), _NEG, dtype=jnp.float32)
    out = _rpe(neg, tab)
    return out[:, :, None]
